# Initial kernel scaffold; baseline (speedup 1.0000x reference)
import jax
import jax.numpy as jnp
from jax.experimental import pallas as pl


def kernel(edge_embeddings, edge_index, W1, b1, W2, b2):
    raise NotImplementedError("write your pallas kernel here")


# speedup vs baseline: 2.4553x; 2.4553x over previous
import functools

import jax
import jax.numpy as jnp
from jax import lax
from jax.experimental import pallas as pl
from jax.experimental.pallas import tpu as pltpu
from jax.experimental.pallas import tpu_sc as plsc

D = 256
N_NODES = 10000
N_EDGES = 160000
K = 50

NC = 2
NS = 16
NW = NC * NS
E_PER_W = N_EDGES // NW
E_BLK = 200
N_CHUNK = E_PER_W // E_BLK
SPLITS = ((0, 96), (96, 104))
N_GROUP = E_BLK // 8

MV_BLK = 2000
NPAD = 8

ROWS = N_EDGES // 128
KPAD = 56


def _mm_body(x_ref, w1a_ref, w1b_ref, a_ref, b_ref):
    x = x_ref[...]
    a_ref[...] = jnp.dot(x, w1a_ref[...], preferred_element_type=jnp.float32)
    b_ref[...] = jnp.dot(x, w1b_ref[...], preferred_element_type=jnp.float32)


def _node_tables(emb, w1a, w1b):
    return pl.pallas_call(
        _mm_body,
        grid=(10,),
        in_specs=[
            pl.BlockSpec((1000, D), lambda i: (i, 0)),
            pl.BlockSpec((D, D), lambda i: (0, 0)),
            pl.BlockSpec((D, D), lambda i: (0, 0)),
        ],
        out_specs=[
            pl.BlockSpec((1000, D), lambda i: (i, 0)),
            pl.BlockSpec((1000, D), lambda i: (i, 0)),
        ],
        out_shape=[jax.ShapeDtypeStruct((N_NODES, D), jnp.float32)] * 2,
    )(emb, w1a, w1b)


def _gather_body(a_hbm, b_hbm, src_hbm, dst_hbm, b1_hbm, h_hbm,
                 idxs_v, idxd_v, rowsa_v, rowsb_v, b1_v, sem):
    wid = lax.axis_index("s") * NC + lax.axis_index("c")
    pltpu.sync_copy(b1_hbm, b1_v)
    b1c = [b1_v[pl.ds(i * 16, 16)] for i in range(16)]

    def chunk_body(c, carry):
        base = wid * E_PER_W + c * E_BLK
        pltpu.sync_copy(src_hbm.at[pl.ds(base, E_BLK)], idxs_v)
        pltpu.sync_copy(dst_hbm.at[pl.ds(base, E_BLK)], idxd_v)
        cps = []
        for (off, ln) in SPLITS:
            sl = pl.ds(off, ln)
            cps.append(pltpu.async_copy(a_hbm.at[idxs_v.at[sl]], rowsa_v.at[sl], sem))
            cps.append(pltpu.async_copy(b_hbm.at[idxd_v.at[sl]], rowsb_v.at[sl], sem))
        for cp in cps:
            cp.wait()

        def row_body(e, carry2):
            for cc in range(16):
                av = rowsa_v[e, pl.ds(cc * 16, 16)]
                bv = rowsb_v[e, pl.ds(cc * 16, 16)]
                rowsa_v[e, pl.ds(cc * 16, 16)] = (av + bv) + b1c[cc]
            return carry2

        lax.fori_loop(0, E_BLK, row_body, 0)
        pltpu.sync_copy(rowsa_v, h_hbm.at[pl.ds(base, E_BLK)])
        return carry

    lax.fori_loop(0, N_CHUNK, chunk_body, 0)


def _edge_hidden(a_tab, b_tab, src, dst, b1):
    mesh = plsc.VectorSubcoreMesh(core_axis_name="c", subcore_axis_name="s")
    fn = functools.partial(
        pl.kernel,
        mesh=mesh,
        compiler_params=pltpu.CompilerParams(needs_layout_passes=False),
        out_type=jax.ShapeDtypeStruct((N_EDGES, D), jnp.float32),
        scratch_types=[
            pltpu.VMEM((E_BLK,), jnp.int32),
            pltpu.VMEM((E_BLK,), jnp.int32),
            pltpu.VMEM((E_BLK, D), jnp.float32),
            pltpu.VMEM((E_BLK, D), jnp.float32),
            pltpu.VMEM((D,), jnp.float32),
            pltpu.SemaphoreType.DMA,
        ],
    )(_gather_body)
    return fn(a_tab, b_tab, src, dst, b1)


def _mv_body(h_ref, w2_ref, b2_ref, o_ref):
    z = jnp.dot(jnp.maximum(h_ref[...], 0.0), w2_ref[...],
                preferred_element_type=jnp.float32) + b2_ref[...]
    o_ref[...] = (1.0 / (1.0 + jnp.exp(-z)))[:, 0:1]


def _edge_scores(h, w2p, b2p):
    return pl.pallas_call(
        _mv_body,
        grid=(N_EDGES // MV_BLK,),
        in_specs=[
            pl.BlockSpec((MV_BLK, D), lambda i: (i, 0)),
            pl.BlockSpec((D, NPAD), lambda i: (0, 0)),
            pl.BlockSpec((1, NPAD), lambda i: (0, 0)),
        ],
        out_specs=pl.BlockSpec((MV_BLK, 1), lambda i: (i, 0)),
        out_shape=jax.ShapeDtypeStruct((N_EDGES, 1), jnp.float32),
    )(h, w2p, b2p)


def _topk_body(sc_ref, ei_ref, os_ref, op_ref, scratch_ref, linid_ref):
    valid = ei_ref[:ROWS, :] < ei_ref[ROWS:, :]
    scratch_ref[...] = jnp.where(valid, sc_ref[...], -jnp.inf)
    linid_ref[...] = (lax.broadcasted_iota(jnp.int32, (ROWS, 128), 0) * 128
                      + lax.broadcasted_iota(jnp.int32, (ROWS, 128), 1))
    col = lax.broadcasted_iota(jnp.int32, (1, 128), 1)
    taken = jnp.int32(2 ** 30)

    def body(kk, carry):
        sv = scratch_ref[...]
        lin = linid_ref[...]
        m = jnp.max(sv)
        idxs = jnp.where(sv == m, lin, taken)
        imin = jnp.min(idxs)
        hit = lin == imin
        scratch_ref[...] = jnp.where(hit, -jnp.inf, sv)
        linid_ref[...] = jnp.where(hit, taken, lin)
        r = imin // 128
        cc = imin % 128
        srow = ei_ref[pl.ds(r, 1), :]
        drow = ei_ref[pl.ds(ROWS + r, 1), :]
        sval = jnp.sum(jnp.where(col == cc, srow, 0))
        dval = jnp.sum(jnp.where(col == cc, drow, 0))
        os_ref[pl.ds(kk, 1), :] = jnp.full((1, 128), m, jnp.float32)
        op_ref[pl.ds(kk, 1), :] = jnp.full((1, 128), sval, jnp.int32)
        op_ref[pl.ds(KPAD + kk, 1), :] = jnp.full((1, 128), dval, jnp.int32)
        return carry

    lax.fori_loop(0, K, body, 0)


def _topk(scores, edge_index):
    return pl.pallas_call(
        _topk_body,
        out_shape=[
            jax.ShapeDtypeStruct((KPAD, 128), jnp.float32),
            jax.ShapeDtypeStruct((2 * KPAD, 128), jnp.int32),
        ],
        scratch_shapes=[pltpu.VMEM((ROWS, 128), jnp.float32),
                        pltpu.VMEM((ROWS, 128), jnp.int32)],
    )(scores.reshape(ROWS, 128), edge_index.reshape(2 * ROWS, 128))


def kernel(edge_embeddings, edge_index, W1, b1, W2, b2):
    a_tab, b_tab = _node_tables(edge_embeddings, W1[:D], W1[D:])
    src = edge_index[0]
    dst = edge_index[1]
    h = _edge_hidden(a_tab, b_tab, src, dst, b1)
    w2p = jnp.concatenate([W2, jnp.zeros((D, NPAD - 1), jnp.float32)], axis=1)
    b2p = jnp.full((1, NPAD), b2[0], jnp.float32)
    scores = _edge_scores(h, w2p, b2p)
    os_pad, op_pad = _topk(scores, edge_index)
    top_scores = os_pad[:K, 0]
    top_pairs = jnp.stack([op_pad[:K, 0], op_pad[KPAD:KPAD + K, 0]], axis=1)
    return (top_pairs, top_scores)


# speedup vs baseline: 2.5713x; 1.0472x over previous
import functools

import jax
import jax.numpy as jnp
from jax import lax
from jax.experimental import pallas as pl
from jax.experimental.pallas import tpu as pltpu
from jax.experimental.pallas import tpu_sc as plsc

D = 256
N_NODES = 10000
N_EDGES = 160000
K = 50

NC = 2
NS = 16
NW = NC * NS
E_PER_W = N_EDGES // NW
E_BLK = 40
N_CHUNK = E_PER_W // E_BLK

MV_BLK = 2000
NPAD = 8

ROWS = N_EDGES // 128
KPAD = 56


def _mm_body(x_ref, w1a_ref, w1b_ref, a_ref, b_ref):
    x = x_ref[...]
    a_ref[...] = jnp.dot(x, w1a_ref[...], preferred_element_type=jnp.float32)
    b_ref[...] = jnp.dot(x, w1b_ref[...], preferred_element_type=jnp.float32)


def _node_tables(emb, w1a, w1b):
    return pl.pallas_call(
        _mm_body,
        grid=(10,),
        in_specs=[
            pl.BlockSpec((1000, D), lambda i: (i, 0)),
            pl.BlockSpec((D, D), lambda i: (0, 0)),
            pl.BlockSpec((D, D), lambda i: (0, 0)),
        ],
        out_specs=[
            pl.BlockSpec((1000, D), lambda i: (i, 0)),
            pl.BlockSpec((1000, D), lambda i: (i, 0)),
        ],
        out_shape=[jax.ShapeDtypeStruct((N_NODES, D), jnp.float32)] * 2,
    )(emb, w1a, w1b)


def _gather_body(a_hbm, b_hbm, src_hbm, dst_hbm, b1_hbm, h_hbm,
                 idxs0, idxs1, idxd0, idxd1, ra0, ra1, rb0, rb1, b1_v,
                 sem0, sem1):
    wid = lax.axis_index("s") * NC + lax.axis_index("c")
    pltpu.sync_copy(b1_hbm, b1_v)
    b1c = [b1_v[pl.ds(i * 16, 16)] for i in range(16)]
    bufs = ((idxs0, idxd0, ra0, rb0, sem0), (idxs1, idxd1, ra1, rb1, sem1))

    def issue(c, p):
        idxs, idxd, ra, rb, sem = bufs[p]
        base = wid * E_PER_W + c * E_BLK
        pltpu.sync_copy(src_hbm.at[pl.ds(base, E_BLK)], idxs)
        pltpu.sync_copy(dst_hbm.at[pl.ds(base, E_BLK)], idxd)
        pltpu.async_copy(a_hbm.at[idxs], ra, sem)
        pltpu.async_copy(b_hbm.at[idxd], rb, sem)

    def finish(c, p):
        idxs, idxd, ra, rb, sem = bufs[p]
        pltpu.make_async_copy(a_hbm.at[pl.ds(0, E_BLK)], ra, sem).wait()
        pltpu.make_async_copy(b_hbm.at[pl.ds(0, E_BLK)], rb, sem).wait()

        def row_body(e, carry2):
            for cc in range(16):
                av = ra[e, pl.ds(cc * 16, 16)]
                bv = rb[e, pl.ds(cc * 16, 16)]
                ra[e, pl.ds(cc * 16, 16)] = (av + bv) + b1c[cc]
            return carry2

        lax.fori_loop(0, E_BLK, row_body, 0)
        base = wid * E_PER_W + c * E_BLK
        pltpu.sync_copy(ra, h_hbm.at[pl.ds(base, E_BLK)])

    issue(0, 0)

    def pair_body(g, carry):
        for b in range(2):
            c = 2 * g + b
            issue(c + 1, 1 - b)
            finish(c, b)
        return carry

    lax.fori_loop(0, (N_CHUNK - 1) // 2, pair_body, 0)
    finish(N_CHUNK - 1, 0)


def _edge_hidden(a_tab, b_tab, src, dst, b1):
    mesh = plsc.VectorSubcoreMesh(core_axis_name="c", subcore_axis_name="s")
    fn = functools.partial(
        pl.kernel,
        mesh=mesh,
        compiler_params=pltpu.CompilerParams(needs_layout_passes=False),
        out_type=jax.ShapeDtypeStruct((N_EDGES, D), jnp.float32),
        scratch_types=[
            pltpu.VMEM((E_BLK,), jnp.int32),
            pltpu.VMEM((E_BLK,), jnp.int32),
            pltpu.VMEM((E_BLK,), jnp.int32),
            pltpu.VMEM((E_BLK,), jnp.int32),
            pltpu.VMEM((E_BLK, D), jnp.float32),
            pltpu.VMEM((E_BLK, D), jnp.float32),
            pltpu.VMEM((E_BLK, D), jnp.float32),
            pltpu.VMEM((E_BLK, D), jnp.float32),
            pltpu.VMEM((D,), jnp.float32),
            pltpu.SemaphoreType.DMA,
            pltpu.SemaphoreType.DMA,
        ],
    )(_gather_body)
    return fn(a_tab, b_tab, src, dst, b1)


def _mv_body(h_ref, w2_ref, b2_ref, o_ref):
    z = jnp.dot(jnp.maximum(h_ref[...], 0.0), w2_ref[...],
                preferred_element_type=jnp.float32) + b2_ref[...]
    o_ref[...] = (1.0 / (1.0 + jnp.exp(-z)))[:, 0:1]


def _edge_scores(h, w2p, b2p):
    return pl.pallas_call(
        _mv_body,
        grid=(N_EDGES // MV_BLK,),
        in_specs=[
            pl.BlockSpec((MV_BLK, D), lambda i: (i, 0)),
            pl.BlockSpec((D, NPAD), lambda i: (0, 0)),
            pl.BlockSpec((1, NPAD), lambda i: (0, 0)),
        ],
        out_specs=pl.BlockSpec((MV_BLK, 1), lambda i: (i, 0)),
        out_shape=jax.ShapeDtypeStruct((N_EDGES, 1), jnp.float32),
    )(h, w2p, b2p)


def _topk_body(sc_ref, ei_ref, os_ref, op_ref, scratch_ref, linid_ref):
    valid = ei_ref[:ROWS, :] < ei_ref[ROWS:, :]
    scratch_ref[...] = jnp.where(valid, sc_ref[...], -jnp.inf)
    linid_ref[...] = (lax.broadcasted_iota(jnp.int32, (ROWS, 128), 0) * 128
                      + lax.broadcasted_iota(jnp.int32, (ROWS, 128), 1))
    col = lax.broadcasted_iota(jnp.int32, (1, 128), 1)
    taken = jnp.int32(2 ** 30)

    def body(kk, carry):
        sv = scratch_ref[...]
        lin = linid_ref[...]
        m = jnp.max(sv)
        idxs = jnp.where(sv == m, lin, taken)
        imin = jnp.min(idxs)
        hit = lin == imin
        scratch_ref[...] = jnp.where(hit, -jnp.inf, sv)
        linid_ref[...] = jnp.where(hit, taken, lin)
        r = imin // 128
        cc = imin % 128
        srow = ei_ref[pl.ds(r, 1), :]
        drow = ei_ref[pl.ds(ROWS + r, 1), :]
        sval = jnp.sum(jnp.where(col == cc, srow, 0))
        dval = jnp.sum(jnp.where(col == cc, drow, 0))
        os_ref[pl.ds(kk, 1), :] = jnp.full((1, 128), m, jnp.float32)
        op_ref[pl.ds(kk, 1), :] = jnp.full((1, 128), sval, jnp.int32)
        op_ref[pl.ds(KPAD + kk, 1), :] = jnp.full((1, 128), dval, jnp.int32)
        return carry

    lax.fori_loop(0, K, body, 0)


def _topk(scores, edge_index):
    return pl.pallas_call(
        _topk_body,
        out_shape=[
            jax.ShapeDtypeStruct((KPAD, 128), jnp.float32),
            jax.ShapeDtypeStruct((2 * KPAD, 128), jnp.int32),
        ],
        scratch_shapes=[pltpu.VMEM((ROWS, 128), jnp.float32),
                        pltpu.VMEM((ROWS, 128), jnp.int32)],
    )(scores.reshape(ROWS, 128), edge_index.reshape(2 * ROWS, 128))


def kernel(edge_embeddings, edge_index, W1, b1, W2, b2):
    a_tab, b_tab = _node_tables(edge_embeddings, W1[:D], W1[D:])
    src = edge_index[0]
    dst = edge_index[1]
    h = _edge_hidden(a_tab, b_tab, src, dst, b1)
    w2p = jnp.concatenate([W2, jnp.zeros((D, NPAD - 1), jnp.float32)], axis=1)
    b2p = jnp.full((1, NPAD), b2[0], jnp.float32)
    scores = _edge_scores(h, w2p, b2p)
    os_pad, op_pad = _topk(scores, edge_index)
    top_scores = os_pad[:K, 0]
    top_pairs = jnp.stack([op_pad[:K, 0], op_pad[KPAD:KPAD + K, 0]], axis=1)
    return (top_pairs, top_scores)


# speedup vs baseline: 2.7249x; 1.0597x over previous
import functools

import jax
import jax.numpy as jnp
from jax import lax
from jax.experimental import pallas as pl
from jax.experimental.pallas import tpu as pltpu
from jax.experimental.pallas import tpu_sc as plsc

D = 256
N_NODES = 10000
N_EDGES = 160000
K = 50

NC = 2
NS = 16
NW = NC * NS
E_PER_W = N_EDGES // NW
E_BLK = 40
N_CHUNK = E_PER_W // E_BLK

MV_BLK = 8000
NPAD = 8

ROWS = N_EDGES // 128
KPAD = 56


def _mm_body(x_ref, w1a_ref, w1b_ref, a_ref, b_ref):
    x = x_ref[...]
    a_ref[...] = jnp.dot(x, w1a_ref[...], preferred_element_type=jnp.float32)
    b_ref[...] = jnp.dot(x, w1b_ref[...], preferred_element_type=jnp.float32)


def _node_tables(emb, w1a, w1b):
    return pl.pallas_call(
        _mm_body,
        grid=(10,),
        in_specs=[
            pl.BlockSpec((1000, D), lambda i: (i, 0)),
            pl.BlockSpec((D, D), lambda i: (0, 0)),
            pl.BlockSpec((D, D), lambda i: (0, 0)),
        ],
        out_specs=[
            pl.BlockSpec((1000, D), lambda i: (i, 0)),
            pl.BlockSpec((1000, D), lambda i: (i, 0)),
        ],
        out_shape=[jax.ShapeDtypeStruct((N_NODES, D), jnp.float32)] * 2,
    )(emb, w1a, w1b)


def _gather_body(a_hbm, b_hbm, src_hbm, dst_hbm, b1_hbm, h_hbm,
                 idxs0, idxs1, idxd0, idxd1, ra0, ra1, rb0, rb1, b1_v,
                 sem0, sem1):
    wid = lax.axis_index("s") * NC + lax.axis_index("c")
    pltpu.sync_copy(b1_hbm, b1_v)
    b1c = [b1_v[pl.ds(i * 16, 16)] for i in range(16)]
    bufs = ((idxs0, idxd0, ra0, rb0, sem0), (idxs1, idxd1, ra1, rb1, sem1))

    def issue(c, p):
        idxs, idxd, ra, rb, sem = bufs[p]
        base = wid * E_PER_W + c * E_BLK
        pltpu.sync_copy(src_hbm.at[pl.ds(base, E_BLK)], idxs)
        pltpu.sync_copy(dst_hbm.at[pl.ds(base, E_BLK)], idxd)
        pltpu.async_copy(a_hbm.at[idxs], ra, sem)
        pltpu.async_copy(b_hbm.at[idxd], rb, sem)

    def finish(c, p):
        idxs, idxd, ra, rb, sem = bufs[p]
        pltpu.make_async_copy(a_hbm.at[pl.ds(0, E_BLK)], ra, sem).wait()
        pltpu.make_async_copy(b_hbm.at[pl.ds(0, E_BLK)], rb, sem).wait()

        def row_body(e, carry2):
            for cc in range(16):
                av = ra[e, pl.ds(cc * 16, 16)]
                bv = rb[e, pl.ds(cc * 16, 16)]
                ra[e, pl.ds(cc * 16, 16)] = (av + bv) + b1c[cc]
            return carry2

        lax.fori_loop(0, E_BLK, row_body, 0)
        base = wid * E_PER_W + c * E_BLK
        pltpu.sync_copy(ra, h_hbm.at[pl.ds(base, E_BLK)])

    issue(0, 0)

    def pair_body(g, carry):
        for b in range(2):
            c = 2 * g + b
            issue(c + 1, 1 - b)
            finish(c, b)
        return carry

    lax.fori_loop(0, (N_CHUNK - 1) // 2, pair_body, 0)
    finish(N_CHUNK - 1, 0)


def _edge_hidden(a_tab, b_tab, src, dst, b1):
    mesh = plsc.VectorSubcoreMesh(core_axis_name="c", subcore_axis_name="s")
    fn = functools.partial(
        pl.kernel,
        mesh=mesh,
        compiler_params=pltpu.CompilerParams(needs_layout_passes=False),
        out_type=jax.ShapeDtypeStruct((N_EDGES, D), jnp.float32),
        scratch_types=[
            pltpu.VMEM((E_BLK,), jnp.int32),
            pltpu.VMEM((E_BLK,), jnp.int32),
            pltpu.VMEM((E_BLK,), jnp.int32),
            pltpu.VMEM((E_BLK,), jnp.int32),
            pltpu.VMEM((E_BLK, D), jnp.float32),
            pltpu.VMEM((E_BLK, D), jnp.float32),
            pltpu.VMEM((E_BLK, D), jnp.float32),
            pltpu.VMEM((E_BLK, D), jnp.float32),
            pltpu.VMEM((D,), jnp.float32),
            pltpu.SemaphoreType.DMA,
            pltpu.SemaphoreType.DMA,
        ],
    )(_gather_body)
    return fn(a_tab, b_tab, src, dst, b1)


def _mv_body(h_ref, w2_ref, b2_ref, o_ref):
    z = jnp.dot(jnp.maximum(h_ref[...], 0.0), w2_ref[...],
                preferred_element_type=jnp.float32) + b2_ref[...]
    o_ref[...] = (1.0 / (1.0 + jnp.exp(-z)))[:, 0:1]


def _edge_scores(h, w2p, b2p):
    return pl.pallas_call(
        _mv_body,
        grid=(N_EDGES // MV_BLK,),
        in_specs=[
            pl.BlockSpec((MV_BLK, D), lambda i: (i, 0)),
            pl.BlockSpec((D, NPAD), lambda i: (0, 0)),
            pl.BlockSpec((1, NPAD), lambda i: (0, 0)),
        ],
        out_specs=pl.BlockSpec((MV_BLK, 1), lambda i: (i, 0)),
        out_shape=jax.ShapeDtypeStruct((N_EDGES, 1), jnp.float32),
    )(h, w2p, b2p)


def _topk_body(sc_ref, ei_ref, os_ref, op_ref, scratch_ref, linid_ref):
    valid = ei_ref[:ROWS, :] < ei_ref[ROWS:, :]
    scratch_ref[...] = jnp.where(valid, sc_ref[...], -jnp.inf)
    linid_ref[...] = (lax.broadcasted_iota(jnp.int32, (ROWS, 128), 0) * 128
                      + lax.broadcasted_iota(jnp.int32, (ROWS, 128), 1))
    col = lax.broadcasted_iota(jnp.int32, (1, 128), 1)
    taken = jnp.int32(2 ** 30)

    def body(kk, carry):
        sv = scratch_ref[...]
        lin = linid_ref[...]
        m = jnp.max(sv)
        idxs = jnp.where(sv == m, lin, taken)
        imin = jnp.min(idxs)
        hit = lin == imin
        scratch_ref[...] = jnp.where(hit, -jnp.inf, sv)
        linid_ref[...] = jnp.where(hit, taken, lin)
        r = imin // 128
        cc = imin % 128
        srow = ei_ref[pl.ds(r, 1), :]
        drow = ei_ref[pl.ds(ROWS + r, 1), :]
        sval = jnp.sum(jnp.where(col == cc, srow, 0))
        dval = jnp.sum(jnp.where(col == cc, drow, 0))
        os_ref[pl.ds(kk, 1), :] = jnp.full((1, 128), m, jnp.float32)
        op_ref[pl.ds(kk, 1), :] = jnp.full((1, 128), sval, jnp.int32)
        op_ref[pl.ds(KPAD + kk, 1), :] = jnp.full((1, 128), dval, jnp.int32)
        return carry

    lax.fori_loop(0, K, body, 0)


def _topk(scores, edge_index):
    return pl.pallas_call(
        _topk_body,
        out_shape=[
            jax.ShapeDtypeStruct((KPAD, 128), jnp.float32),
            jax.ShapeDtypeStruct((2 * KPAD, 128), jnp.int32),
        ],
        scratch_shapes=[pltpu.VMEM((ROWS, 128), jnp.float32),
                        pltpu.VMEM((ROWS, 128), jnp.int32)],
    )(scores.reshape(ROWS, 128), edge_index.reshape(2 * ROWS, 128))


def kernel(edge_embeddings, edge_index, W1, b1, W2, b2):
    a_tab, b_tab = _node_tables(edge_embeddings, W1[:D], W1[D:])
    src = edge_index[0]
    dst = edge_index[1]
    h = _edge_hidden(a_tab, b_tab, src, dst, b1)
    w2p = jnp.concatenate([W2, jnp.zeros((D, NPAD - 1), jnp.float32)], axis=1)
    b2p = jnp.full((1, NPAD), b2[0], jnp.float32)
    scores = _edge_scores(h, w2p, b2p)
    os_pad, op_pad = _topk(scores, edge_index)
    top_scores = os_pad[:K, 0]
    top_pairs = jnp.stack([op_pad[:K, 0], op_pad[KPAD:KPAD + K, 0]], axis=1)
    return (top_pairs, top_scores)


# speedup vs baseline: 2.7380x; 1.0048x over previous
import functools

import jax
import jax.numpy as jnp
from jax import lax
from jax.experimental import pallas as pl
from jax.experimental.pallas import tpu as pltpu
from jax.experimental.pallas import tpu_sc as plsc

D = 256
N_NODES = 10000
N_EDGES = 160000
K = 50

NC = 2
NS = 16
NW = NC * NS
E_PER_W = N_EDGES // NW
E_BLK = 40
N_CHUNK = E_PER_W // E_BLK

MV_BLK = 8000
NPAD = 8

ROWS = N_EDGES // 128
KPAD = 56


def _mm_body(x_ref, w1a_ref, w1b_ref, a_ref, b_ref):
    x = x_ref[...]
    a_ref[...] = jnp.dot(x, w1a_ref[...], preferred_element_type=jnp.float32)
    b_ref[...] = jnp.dot(x, w1b_ref[...], preferred_element_type=jnp.float32)


def _node_tables(emb, w1a, w1b):
    return pl.pallas_call(
        _mm_body,
        grid=(10,),
        in_specs=[
            pl.BlockSpec((1000, D), lambda i: (i, 0)),
            pl.BlockSpec((D, D), lambda i: (0, 0)),
            pl.BlockSpec((D, D), lambda i: (0, 0)),
        ],
        out_specs=[
            pl.BlockSpec((1000, D), lambda i: (i, 0)),
            pl.BlockSpec((1000, D), lambda i: (i, 0)),
        ],
        out_shape=[jax.ShapeDtypeStruct((N_NODES, D), jnp.float32)] * 2,
    )(emb, w1a, w1b)


def _gather_body(a_hbm, b_hbm, src_hbm, dst_hbm, b1_hbm, h_hbm,
                 idxs0, idxs1, idxd0, idxd1, ra0, ra1, rb0, rb1, b1_v,
                 sem0, sem1):
    wid = lax.axis_index("s") * NC + lax.axis_index("c")
    pltpu.sync_copy(b1_hbm, b1_v)
    b1c = [b1_v[pl.ds(i * 16, 16)] for i in range(16)]
    bufs = ((idxs0, idxd0, ra0, rb0, sem0), (idxs1, idxd1, ra1, rb1, sem1))

    def issue(c, p):
        idxs, idxd, ra, rb, sem = bufs[p]
        base = wid * E_PER_W + c * E_BLK
        pltpu.sync_copy(src_hbm.at[pl.ds(base, E_BLK)], idxs)
        pltpu.sync_copy(dst_hbm.at[pl.ds(base, E_BLK)], idxd)
        pltpu.async_copy(a_hbm.at[idxs], ra, sem)
        pltpu.async_copy(b_hbm.at[idxd], rb, sem)

    def finish(c, p):
        idxs, idxd, ra, rb, sem = bufs[p]
        pltpu.make_async_copy(a_hbm.at[pl.ds(0, E_BLK)], ra, sem).wait()
        pltpu.make_async_copy(b_hbm.at[pl.ds(0, E_BLK)], rb, sem).wait()

        def row_body(e, carry2):
            for cc in range(16):
                av = ra[e, pl.ds(cc * 16, 16)]
                bv = rb[e, pl.ds(cc * 16, 16)]
                ra[e, pl.ds(cc * 16, 16)] = (av + bv) + b1c[cc]
            return carry2

        lax.fori_loop(0, E_BLK, row_body, 0)
        base = wid * E_PER_W + c * E_BLK
        pltpu.sync_copy(ra, h_hbm.at[pl.ds(base, E_BLK)])

    issue(0, 0)

    def pair_body(g, carry):
        for b in range(2):
            c = 2 * g + b
            issue(c + 1, 1 - b)
            finish(c, b)
        return carry

    lax.fori_loop(0, (N_CHUNK - 1) // 2, pair_body, 0)
    finish(N_CHUNK - 1, 0)


def _edge_hidden(a_tab, b_tab, src, dst, b1):
    mesh = plsc.VectorSubcoreMesh(core_axis_name="c", subcore_axis_name="s")
    fn = functools.partial(
        pl.kernel,
        mesh=mesh,
        compiler_params=pltpu.CompilerParams(needs_layout_passes=False),
        out_type=jax.ShapeDtypeStruct((N_EDGES, D), jnp.float32),
        scratch_types=[
            pltpu.VMEM((E_BLK,), jnp.int32),
            pltpu.VMEM((E_BLK,), jnp.int32),
            pltpu.VMEM((E_BLK,), jnp.int32),
            pltpu.VMEM((E_BLK,), jnp.int32),
            pltpu.VMEM((E_BLK, D), jnp.float32),
            pltpu.VMEM((E_BLK, D), jnp.float32),
            pltpu.VMEM((E_BLK, D), jnp.float32),
            pltpu.VMEM((E_BLK, D), jnp.float32),
            pltpu.VMEM((D,), jnp.float32),
            pltpu.SemaphoreType.DMA,
            pltpu.SemaphoreType.DMA,
        ],
    )(_gather_body)
    return fn(a_tab, b_tab, src, dst, b1)


def _mv_body(h_ref, w2_ref, b2_ref, o_ref):
    z = jnp.dot(jnp.maximum(h_ref[...], 0.0), w2_ref[...],
                preferred_element_type=jnp.float32) + b2_ref[...]
    o_ref[...] = (1.0 / (1.0 + jnp.exp(-z)))[:, 0:1]


def _edge_scores(h, w2p, b2p):
    return pl.pallas_call(
        _mv_body,
        grid=(N_EDGES // MV_BLK,),
        in_specs=[
            pl.BlockSpec((MV_BLK, D), lambda i: (i, 0)),
            pl.BlockSpec((D, NPAD), lambda i: (0, 0)),
            pl.BlockSpec((1, NPAD), lambda i: (0, 0)),
        ],
        out_specs=pl.BlockSpec((MV_BLK, 1), lambda i: (i, 0)),
        out_shape=jax.ShapeDtypeStruct((N_EDGES, 1), jnp.float32),
    )(h, w2p, b2p)


def _topk_body(sc_ref, ei_ref, os_ref, op_ref, scratch_ref, linid_ref):
    valid = ei_ref[:ROWS, :] < ei_ref[ROWS:, :]
    scratch_ref[...] = jnp.where(valid, sc_ref[...], -jnp.inf)
    linid_ref[...] = (lax.broadcasted_iota(jnp.int32, (ROWS, 128), 0) * 128
                      + lax.broadcasted_iota(jnp.int32, (ROWS, 128), 1))
    col = lax.broadcasted_iota(jnp.int32, (1, 128), 1)
    taken = jnp.int32(2 ** 30)

    def body(kk, carry):
        sv = scratch_ref[...]
        lin = linid_ref[...]
        m = jnp.max(sv)
        idxs = jnp.where(sv == m, lin, taken)
        imin = jnp.min(idxs)
        r = imin // 128
        cc = imin % 128
        rowmask = col == cc
        hrow = scratch_ref[pl.ds(r, 1), :]
        scratch_ref[pl.ds(r, 1), :] = jnp.where(rowmask, -jnp.inf, hrow)
        lrow = linid_ref[pl.ds(r, 1), :]
        linid_ref[pl.ds(r, 1), :] = jnp.where(rowmask, taken, lrow)
        srow = ei_ref[pl.ds(r, 1), :]
        drow = ei_ref[pl.ds(ROWS + r, 1), :]
        sval = jnp.sum(jnp.where(col == cc, srow, 0))
        dval = jnp.sum(jnp.where(col == cc, drow, 0))
        os_ref[pl.ds(kk, 1), :] = jnp.full((1, 128), m, jnp.float32)
        op_ref[pl.ds(kk, 1), :] = jnp.full((1, 128), sval, jnp.int32)
        op_ref[pl.ds(KPAD + kk, 1), :] = jnp.full((1, 128), dval, jnp.int32)
        return carry

    lax.fori_loop(0, K, body, 0)


def _topk(scores, edge_index):
    return pl.pallas_call(
        _topk_body,
        out_shape=[
            jax.ShapeDtypeStruct((KPAD, 128), jnp.float32),
            jax.ShapeDtypeStruct((2 * KPAD, 128), jnp.int32),
        ],
        scratch_shapes=[pltpu.VMEM((ROWS, 128), jnp.float32),
                        pltpu.VMEM((ROWS, 128), jnp.int32)],
    )(scores.reshape(ROWS, 128), edge_index.reshape(2 * ROWS, 128))


def kernel(edge_embeddings, edge_index, W1, b1, W2, b2):
    a_tab, b_tab = _node_tables(edge_embeddings, W1[:D], W1[D:])
    src = edge_index[0]
    dst = edge_index[1]
    h = _edge_hidden(a_tab, b_tab, src, dst, b1)
    w2p = jnp.concatenate([W2, jnp.zeros((D, NPAD - 1), jnp.float32)], axis=1)
    b2p = jnp.full((1, NPAD), b2[0], jnp.float32)
    scores = _edge_scores(h, w2p, b2p)
    os_pad, op_pad = _topk(scores, edge_index)
    top_scores = os_pad[:K, 0]
    top_pairs = jnp.stack([op_pad[:K, 0], op_pad[KPAD:KPAD + K, 0]], axis=1)
    return (top_pairs, top_scores)
